# baseline (speedup 1.0000x reference)
import functools
import math

import jax
import jax.numpy as jnp
from jax import lax
from jax.experimental import pallas as pl
from jax.experimental.pallas import tpu as pltpu
from jax.experimental.pallas import tpu_sc as plsc

_D = 128
_S = 200
_B = 1024
_NC = 2
_NS = 16
_NW = _NC * _NS
_SPW = _B // _NW
_T = 4
_NCH = _S // _T
_CR = _SPW * _T
_NB = 3
_SCALE = math.sqrt(float(_D))

_mesh = plsc.VectorSubcoreMesh(core_axis_name="c", subcore_axis_name="s")


@functools.partial(
    pl.kernel,
    mesh=_mesh,
    out_type=jax.ShapeDtypeStruct((_B, _S, _D), jnp.float32),
    scratch_types=[
        pltpu.VMEM((_NCH * _CR,), jnp.int32),
    ]
    + [pltpu.VMEM((_CR, _D), jnp.float32) for _ in range(_NB)]
    + [pltpu.VMEM((_SPW, _T, _D), jnp.float32) for _ in range(_NB)]
    + [pltpu.VMEM((_T, _D), jnp.float32) for _ in range(_NB)]
    + [pltpu.SemaphoreType.DMA for _ in range(2 * _NB)],
)
def _sc_embed(idx_hbm, p_hbm, table_hbm, out_hbm, idx_v, *rest):
    gbufs = rest[:_NB]
    obufs = rest[_NB:2 * _NB]
    pbufs = rest[2 * _NB:3 * _NB]
    gsems = rest[3 * _NB:4 * _NB]
    osems = rest[4 * _NB:]
    wid = lax.axis_index("s") * _NC + lax.axis_index("c")
    sbase = wid * _SPW
    pltpu.sync_copy(idx_hbm.at[pl.ds(wid * _NCH * _CR, _NCH * _CR)], idx_v)

    def gather_issue(tc, b):
        pltpu.async_copy(
            table_hbm.at[idx_v.at[pl.ds(tc * _CR, _CR)]], gbufs[b], gsems[b]
        )
        pltpu.async_copy(p_hbm.at[pl.ds(tc * _T, _T)], pbufs[b], gsems[b])

    def gather_wait(b):
        pltpu.make_async_copy(
            table_hbm.at[idx_v.at[pl.ds(0, _CR)]], gbufs[b], gsems[b]
        ).wait()
        pltpu.make_async_copy(
            p_hbm.at[pl.ds(0, _T)], pbufs[b], gsems[b]
        ).wait()

    def out_issue(tc, b):
        pltpu.async_copy(
            obufs[b],
            out_hbm.at[pl.ds(sbase, _SPW), pl.ds(tc * _T, _T)],
            osems[b],
        )

    def out_wait(b):
        pltpu.make_async_copy(
            obufs[b],
            out_hbm.at[pl.ds(0, _SPW), pl.ds(0, _T)],
            osems[b],
        ).wait()

    def compute(b):
        for i in range(_T):
            pv = [pbufs[b][i, pl.ds(cc * 16, 16)] for cc in range(_D // 16)]

            def s_body(s, c2, _b=b, _i=i, _pv=pv):
                for cc in range(_D // 16):
                    sl = pl.ds(cc * 16, 16)
                    obufs[_b][s, _i, sl] = (
                        gbufs[_b][s * _T + _i, sl] * _SCALE + _pv[cc]
                    )
                return c2

            lax.fori_loop(0, _SPW, s_body, 0, unroll=2)

    for b in range(_NB - 1):
        gather_issue(b, b)

    def step(j, carry, b):
        gather_wait(b)
        g = j + _NB - 1

        @pl.when(g < _NCH)
        def _():
            gather_issue(g, (b + _NB - 1) % _NB)

        @pl.when(j >= _NB)
        def _():
            out_wait(b)

        compute(b)
        out_issue(j, b)
        return carry

    def outer(o, carry):
        for b in range(_NB):
            step(o * _NB + b, carry, b)
        return carry

    lax.fori_loop(0, _NCH // _NB, outer, 0, unroll=False)
    for j in range((_NCH // _NB) * _NB, _NCH):
        step(j, 0, j % _NB)
    for b in range(_NB):
        out_wait(b)


def kernel(seq, embedding, p):
    idx_r = (
        seq.astype(jnp.int32)
        .reshape(_NW, _SPW, _NCH, _T)
        .transpose(0, 2, 1, 3)
        .reshape(-1)
    )
    return _sc_embed(idx_r, p[:_S], embedding)


# speedup vs baseline: 1.3515x; 1.3515x over previous
import functools
import math

import jax
import jax.numpy as jnp
from jax import lax
from jax.experimental import pallas as pl
from jax.experimental.pallas import tpu as pltpu
from jax.experimental.pallas import tpu_sc as plsc

_D = 128
_S = 200
_B = 1024
_NC = 2
_NS = 16
_NW = _NC * _NS
_ROWS = _B * _S
_RPW = _ROWS // _NW
_CH = _S
_NCH = _RPW // _CH
_NB = 3
_SCALE = math.sqrt(float(_D))

_mesh = plsc.VectorSubcoreMesh(core_axis_name="c", subcore_axis_name="s")


@functools.partial(
    pl.kernel,
    mesh=_mesh,
    out_type=jax.ShapeDtypeStruct((_ROWS, _D), jnp.float32),
    scratch_types=[
        pltpu.VMEM((_RPW,), jnp.int32),
        pltpu.VMEM((_S, _D), jnp.float32),
    ]
    + [pltpu.VMEM((_CH, _D), jnp.float32) for _ in range(_NB)]
    + [pltpu.SemaphoreType.DMA for _ in range(2 * _NB)],
)
def _sc_embed(idx_hbm, p_hbm, table_hbm, out_hbm, idx_v, p_v, *rest):
    bufs = rest[:_NB]
    gsems = rest[_NB:2 * _NB]
    osems = rest[2 * _NB:]
    wid = lax.axis_index("s") * _NC + lax.axis_index("c")
    base = wid * _RPW
    pltpu.sync_copy(idx_hbm.at[pl.ds(base, _RPW)], idx_v)
    pltpu.sync_copy(p_hbm, p_v)

    def gather_issue(j, b):
        pltpu.async_copy(
            table_hbm.at[idx_v.at[pl.ds(j * _CH, _CH)]], bufs[b], gsems[b]
        )

    def gather_wait(b):
        pltpu.make_async_copy(
            table_hbm.at[idx_v.at[pl.ds(0, _CH)]], bufs[b], gsems[b]
        ).wait()

    def out_issue(j, b):
        pltpu.async_copy(
            bufs[b], out_hbm.at[pl.ds(base + j * _CH, _CH)], osems[b]
        )

    def out_wait(b):
        pltpu.make_async_copy(
            bufs[b], out_hbm.at[pl.ds(0, _CH)], osems[b]
        ).wait()

    def compute(b):
        def row_body(r, c2, _b=b):
            for cc in range(_D // 16):
                sl = pl.ds(cc * 16, 16)
                bufs[_b][r, sl] = bufs[_b][r, sl] * _SCALE + p_v[r, sl]
            return c2

        lax.fori_loop(0, _CH, row_body, 0, unroll=4)

    for b in range(_NB - 1):
        gather_issue(b, b)

    def step(j, b):
        gather_wait(b)
        compute(b)
        out_issue(j, b)
        nb = (b + _NB - 1) % _NB
        g = j + _NB - 1

        @pl.when(g < _NCH)
        def _():
            @pl.when(j >= 1)
            def _():
                out_wait(nb)

            gather_issue(g, nb)

    def outer(o, carry):
        for b in range(_NB):
            step(o * _NB + b, b)
        return carry

    lax.fori_loop(0, _NCH // _NB, outer, 0, unroll=False)
    for j in range((_NCH // _NB) * _NB, _NCH):
        step(j, j % _NB)
    for b in range(_NB):
        out_wait(b)


def kernel(seq, embedding, p):
    idx = seq.reshape(-1).astype(jnp.int32)
    out = _sc_embed(idx, p[:_S], embedding)
    return out.reshape(_B, _S, _D)


# speedup vs baseline: 3.4362x; 2.5425x over previous
import functools
import math

import jax
import jax.numpy as jnp
from jax import lax
from jax.experimental import pallas as pl
from jax.experimental.pallas import tpu as pltpu
from jax.experimental.pallas import tpu_sc as plsc

_D = 128
_S = 200
_B = 1024
_NC = 2
_NS = 16
_NW = _NC * _NS
_ROWS = _B * _S
_RPW = _ROWS // _NW
_CH = _S
_NCH = _RPW // _CH
_NB = 3
_SCALE = math.sqrt(float(_D))

_mesh = plsc.VectorSubcoreMesh(core_axis_name="c", subcore_axis_name="s")


@functools.partial(
    pl.kernel,
    mesh=_mesh,
    out_type=jax.ShapeDtypeStruct((_ROWS, _D), jnp.float32),
    scratch_types=[
        pltpu.VMEM((_RPW,), jnp.int32),
        pltpu.VMEM((_S, _D), jnp.float32),
    ]
    + [pltpu.VMEM((_CH, _D), jnp.float32) for _ in range(_NB)]
    + [pltpu.SemaphoreType.DMA for _ in range(2 * _NB)],
)
def _sc_embed(idx_hbm, p_hbm, table_hbm, out_hbm, idx_v, p_v, *rest):
    bufs = rest[:_NB]
    gsems = rest[_NB:2 * _NB]
    osems = rest[2 * _NB:]
    wid = lax.axis_index("s") * _NC + lax.axis_index("c")
    base = wid * _RPW
    pltpu.sync_copy(idx_hbm.at[pl.ds(base, _RPW)], idx_v)
    pltpu.sync_copy(p_hbm, p_v)

    def gather_issue(j, b):
        pltpu.async_copy(
            table_hbm.at[idx_v.at[pl.ds(j * _CH, _CH)]], bufs[b], gsems[b]
        )

    def gather_wait(b):
        pltpu.make_async_copy(
            table_hbm.at[idx_v.at[pl.ds(0, _CH)]], bufs[b], gsems[b]
        ).wait()

    def out_issue(j, b):
        pltpu.async_copy(
            bufs[b], out_hbm.at[pl.ds(base + j * _CH, _CH)], osems[b]
        )

    def out_wait(b):
        pltpu.make_async_copy(
            bufs[b], out_hbm.at[pl.ds(0, _CH)], osems[b]
        ).wait()

    def compute(b):
        @plsc.parallel_loop(0, _CH, unroll=4)
        def row_body(r, _b=b):
            for cc in range(_D // 16):
                sl = pl.ds(cc * 16, 16)
                bufs[_b][r, sl] = bufs[_b][r, sl] * _SCALE + p_v[r, sl]

    for b in range(_NB - 1):
        gather_issue(b, b)

    def step(j, b):
        gather_wait(b)
        compute(b)
        out_issue(j, b)
        nb = (b + _NB - 1) % _NB
        g = j + _NB - 1

        @pl.when(g < _NCH)
        def _():
            @pl.when(j >= 1)
            def _():
                out_wait(nb)

            gather_issue(g, nb)

    def outer(o, carry):
        for b in range(_NB):
            step(o * _NB + b, b)
        return carry

    lax.fori_loop(0, _NCH // _NB, outer, 0, unroll=False)
    for j in range((_NCH // _NB) * _NB, _NCH):
        step(j, j % _NB)
    for b in range(_NB):
        out_wait(b)


def kernel(seq, embedding, p):
    idx = seq.reshape(-1).astype(jnp.int32)
    out = _sc_embed(idx, p[:_S], embedding)
    return out.reshape(_B, _S, _D)


# speedup vs baseline: 3.4762x; 1.0116x over previous
import functools
import math

import jax
import jax.numpy as jnp
from jax import lax
from jax.experimental import pallas as pl
from jax.experimental.pallas import tpu as pltpu
from jax.experimental.pallas import tpu_sc as plsc

_D = 128
_S = 200
_B = 1024
_NC = 2
_NS = 16
_NW = _NC * _NS
_ROWS = _B * _S
_RPW = _ROWS // _NW
_CH = _S
_NCH = _RPW // _CH
_NB = 3
_SCALE = math.sqrt(float(_D))

_mesh = plsc.VectorSubcoreMesh(core_axis_name="c", subcore_axis_name="s")


@functools.partial(
    pl.kernel,
    mesh=_mesh,
    out_type=jax.ShapeDtypeStruct((_ROWS, _D), jnp.float32),
    scratch_types=[
        pltpu.VMEM((_RPW,), jnp.int32),
        pltpu.VMEM((_S, _D), jnp.float32),
    ]
    + [pltpu.VMEM((_CH, _D), jnp.float32) for _ in range(_NB)]
    + [pltpu.VMEM((_CH,), jnp.int32)]
    + [pltpu.SemaphoreType.DMA for _ in range(2 * _NB + 1)],
)
def _sc_embed(idx_hbm, p_hbm, table_hbm, out_hbm, idx_v, p_v, *rest):
    bufs = rest[:_NB]
    idx0_v = rest[_NB]
    gsems = rest[_NB + 1:2 * _NB + 1]
    osems = rest[2 * _NB + 1:3 * _NB + 1]
    ssem = rest[3 * _NB + 1]
    wid = lax.axis_index("s") * _NC + lax.axis_index("c")
    base = wid * _RPW
    pltpu.async_copy(idx_hbm.at[pl.ds(base, _RPW)], idx_v, ssem)
    pltpu.async_copy(p_hbm, p_v, ssem)
    pltpu.sync_copy(idx_hbm.at[pl.ds(base, _CH)], idx0_v)
    pltpu.async_copy(table_hbm.at[idx0_v], bufs[0], gsems[0])
    pltpu.make_async_copy(idx_hbm.at[pl.ds(base, _RPW)], idx_v, ssem).wait()

    def gather_issue(j, b):
        pltpu.async_copy(
            table_hbm.at[idx_v.at[pl.ds(j * _CH, _CH)]], bufs[b], gsems[b]
        )

    def gather_wait(b):
        pltpu.make_async_copy(
            table_hbm.at[idx_v.at[pl.ds(0, _CH)]], bufs[b], gsems[b]
        ).wait()

    def out_issue(j, b):
        pltpu.async_copy(
            bufs[b], out_hbm.at[pl.ds(base + j * _CH, _CH)], osems[b]
        )

    def out_wait(b):
        pltpu.make_async_copy(
            bufs[b], out_hbm.at[pl.ds(0, _CH)], osems[b]
        ).wait()

    def compute(b):
        @plsc.parallel_loop(0, _CH, unroll=4)
        def row_body(r, _b=b):
            for cc in range(_D // 16):
                sl = pl.ds(cc * 16, 16)
                bufs[_b][r, sl] = bufs[_b][r, sl] * _SCALE + p_v[r, sl]

    for b in range(1, _NB - 1):
        gather_issue(b, b)
    pltpu.make_async_copy(p_hbm, p_v, ssem).wait()

    def step(j, b):
        gather_wait(b)
        compute(b)
        out_issue(j, b)
        nb = (b + _NB - 1) % _NB
        g = j + _NB - 1

        @pl.when(g < _NCH)
        def _():
            @pl.when(j >= 1)
            def _():
                out_wait(nb)

            gather_issue(g, nb)

    def outer(o, carry):
        for b in range(_NB):
            step(o * _NB + b, b)
        return carry

    lax.fori_loop(0, _NCH // _NB, outer, 0, unroll=False)
    for j in range((_NCH // _NB) * _NB, _NCH):
        step(j, j % _NB)
    for b in range(_NB):
        out_wait(b)


def kernel(seq, embedding, p):
    idx = seq.reshape(-1).astype(jnp.int32)
    out = _sc_embed(idx, p[:_S], embedding)
    return out.reshape(_B, _S, _D)
